# baseline (speedup 1.0000x reference)
import functools

import jax
import jax.numpy as jnp
from jax.experimental import pallas as pl
from jax.experimental.pallas import tpu as pltpu

_EPS = 1e-5
_VMEM_LIMIT = 48 * 1024 * 1024


def _deint(x_ref):
    xi = pltpu.bitcast(x_ref[...], jnp.int32)
    lo = jax.lax.bitcast_convert_type(xi.astype(jnp.int16), jnp.bfloat16)
    hi = jax.lax.bitcast_convert_type(
        (xi >> 16).astype(jnp.int16), jnp.bfloat16)
    return jnp.concatenate([lo, hi], axis=-1)


def _block_body(x_ref, b1_ref, w1_ref, s2_ref, b2_ref, w2_ref,
                o_ref, *, nb, ho, wo, cin, co):
    m = ho * wo
    f32 = jnp.float32
    c2 = 2 * cin
    bf16 = jnp.bfloat16

    xp = _deint(x_ref)
    a = jnp.maximum(xp.astype(f32) + b1_ref[0], 0.0).astype(bf16)

    ar = a.reshape(nb, ho, 2, wo, c2)
    ev = ar[:, :, 0]
    od = ar[:, :, 1]

    p0 = jnp.concatenate(
        [jnp.zeros((nb, 1, wo, c2), bf16), od[:, :ho - 1]], axis=1)

    pieces = []
    for p in (p0, ev, od):
        shift = jnp.concatenate(
            [jnp.zeros((nb, ho, 1, c2), bf16), p[:, :, :wo - 1]], axis=2)
        pieces.append(p.reshape(nb * m, c2))
        pieces.append(shift.reshape(nb * m, c2))
    acc = jnp.dot(jnp.concatenate(pieces, axis=1), w1_ref[...],
                  preferred_element_type=f32)

    a2 = jnp.maximum(acc * s2_ref[0] + b2_ref[0], 0.0).astype(bf16)
    a2p = jnp.pad(a2.reshape(nb, ho, wo, co),
                  ((0, 0), (1, 1), (1, 1), (0, 0)))

    pieces = []
    for dy in range(3):
        rows = a2p[:, dy:dy + ho]
        for dx in range(3):
            pieces.append(rows[:, :, dx:dx + wo].reshape(nb * m, co))
    pieces.append(ev.reshape(nb * m, c2))
    out = jnp.dot(jnp.concatenate(pieces, axis=1), w2_ref[...],
                  preferred_element_type=f32)
    o_ref[...] = out.reshape(nb, ho, wo, co)


def kernel(x, bn1_gamma, bn1_beta, bn1_mean, bn1_var,
           bn2_gamma, bn2_beta, bn2_mean, bn2_var, w1, w2, w_sc):
    n, cin, h, w = x.shape
    co = w1.shape[0]
    ho, wo = h // 2, w // 2
    nb = 16 if n % 16 == 0 else 1
    bf16 = jnp.bfloat16

    s1 = bn1_gamma / jnp.sqrt(bn1_var + _EPS)
    b1 = bn1_beta - bn1_mean * s1
    s2 = bn2_gamma / jnp.sqrt(bn2_var + _EPS)
    b2 = bn2_beta - bn2_mean * s2

    xnh = x.transpose(0, 2, 3, 1).astype(bf16)

    b1s = b1 / s1
    b1t = jnp.concatenate([b1s, b1s]).reshape(1, 2 * cin).astype(jnp.float32)

    zero = jnp.zeros((3, cin, co), jnp.float32)
    wt = jnp.transpose(w1, (2, 1, 3, 0)) * s1[None, :, None, None]
    w1k = jnp.concatenate(
        [wt[:, :, 1], wt[:, :, 2], zero, wt[:, :, 0]],
        axis=1).reshape(3 * 4 * cin, co).astype(bf16)

    w2t = jnp.transpose(w2, (2, 3, 1, 0)).reshape(9 * co, co)
    w2k = jnp.concatenate(
        [w2t, w_sc.reshape(co, cin).T * s1[:, None], jnp.zeros((cin, co))],
        axis=0).astype(bf16)

    body = functools.partial(_block_body, nb=nb, ho=ho, wo=wo, cin=cin, co=co)
    out = pl.pallas_call(
        body,
        grid=(n // nb,),
        in_specs=[
            pl.BlockSpec((nb, h, w, cin), lambda i: (i, 0, 0, 0)),
            pl.BlockSpec((1, 2 * cin), lambda i: (0, 0)),
            pl.BlockSpec((12 * cin, co), lambda i: (0, 0)),
            pl.BlockSpec((1, co), lambda i: (0, 0)),
            pl.BlockSpec((1, co), lambda i: (0, 0)),
            pl.BlockSpec((9 * co + 2 * cin, co), lambda i: (0, 0)),
        ],
        out_specs=pl.BlockSpec((nb, ho, wo, co), lambda i: (i, 0, 0, 0)),
        out_shape=jax.ShapeDtypeStruct((n, ho, wo, co), jnp.float32),
        compiler_params=pltpu.CompilerParams(
            dimension_semantics=("parallel",),
            vmem_limit_bytes=_VMEM_LIMIT),
        cost_estimate=pl.CostEstimate(
            flops=2 * n * ho * wo * 9 * (cin + co) * co,
            transcendentals=0,
            bytes_accessed=2 * n * h * w * cin + 4 * n * ho * wo * co),
    )(xnh, b1t, w1k, s2.reshape(1, co), b2.reshape(1, co), w2k)

    return jnp.transpose(out, (0, 3, 1, 2))


# speedup vs baseline: 1.0234x; 1.0234x over previous
import functools

import jax
import jax.numpy as jnp
from jax.experimental import pallas as pl
from jax.experimental.pallas import tpu as pltpu

_EPS = 1e-5
_VMEM_LIMIT = 48 * 1024 * 1024


def _one_image(a_all, s2, b2, w1_ref, w2_ref, o_ref, *, ho, wo, cin, co):
    m = ho * wo
    f32 = jnp.float32
    c2 = 2 * cin

    ar = a_all.reshape(ho, 2, wo, c2)
    ev = ar[:, 0]
    od = ar[:, 1]

    p0 = jnp.concatenate([jnp.zeros((1, wo, c2), a_all.dtype),
                          od[:ho - 1]], axis=0)
    planes = (p0, ev, od)

    pieces = []
    for dy in range(3):
        p = planes[dy]
        shift = jnp.concatenate(
            [jnp.zeros((ho, 1, c2), a_all.dtype), p[:, :wo - 1]], axis=1)
        pieces.append(p.reshape(m, c2))
        pieces.append(shift.reshape(m, c2))
    acc = jnp.dot(jnp.concatenate(pieces, axis=1), w1_ref[...],
                  preferred_element_type=f32)

    a2 = jnp.maximum(acc * s2 + b2, 0.0).astype(jnp.bfloat16)
    a2p = jnp.pad(a2.reshape(ho, wo, co), ((1, 1), (1, 1), (0, 0)))

    pieces = []
    for dy in range(3):
        rows = a2p[dy:dy + ho]
        for dx in range(3):
            pieces.append(rows[:, dx:dx + wo].reshape(m, co))
    pieces.append(ev.reshape(m, c2))
    out = jnp.dot(jnp.concatenate(pieces, axis=1), w2_ref[...],
                  preferred_element_type=f32)
    o_ref[...] = out.reshape(ho, wo, co)


def _deint(x_ref):
    xi = pltpu.bitcast(x_ref[...], jnp.int32)
    lo = jax.lax.bitcast_convert_type(xi.astype(jnp.int16), jnp.bfloat16)
    hi = jax.lax.bitcast_convert_type(
        (xi >> 16).astype(jnp.int16), jnp.bfloat16)
    return jnp.concatenate([lo, hi], axis=-1)


def _block_body(x_ref, b1_ref, w1_ref, s2_ref, b2_ref, w2_ref,
                o_ref, *, nb, ho, wo, cin, co):
    xp = _deint(x_ref)

    a = jnp.maximum(xp.astype(jnp.float32) + b1_ref[0], 0.0).astype(jnp.bfloat16)
    s2, b2 = s2_ref[0], b2_ref[0]
    for b in range(nb):
        _one_image(a[b], s2, b2, w1_ref, w2_ref, o_ref.at[b],
                   ho=ho, wo=wo, cin=cin, co=co)


def kernel(x, bn1_gamma, bn1_beta, bn1_mean, bn1_var,
           bn2_gamma, bn2_beta, bn2_mean, bn2_var, w1, w2, w_sc):
    n, cin, h, w = x.shape
    co = w1.shape[0]
    ho, wo = h // 2, w // 2
    nb = 16 if n % 16 == 0 else 1
    bf16 = jnp.bfloat16

    s1 = bn1_gamma / jnp.sqrt(bn1_var + _EPS)
    b1 = bn1_beta - bn1_mean * s1
    s2 = bn2_gamma / jnp.sqrt(bn2_var + _EPS)
    b2 = bn2_beta - bn2_mean * s2

    xnh = x.transpose(0, 2, 3, 1).astype(bf16)

    b1s = b1 / s1
    b1t = jnp.concatenate([b1s, b1s]).reshape(1, 2 * cin).astype(jnp.float32)

    zero = jnp.zeros((3, cin, co), jnp.float32)
    wt = jnp.transpose(w1, (2, 1, 3, 0)) * s1[None, :, None, None]
    w1k = jnp.concatenate(
        [wt[:, :, 1], wt[:, :, 2], zero, wt[:, :, 0]],
        axis=1).reshape(3 * 4 * cin, co).astype(bf16)

    w2t = jnp.transpose(w2, (2, 3, 1, 0)).reshape(9 * co, co)
    w2k = jnp.concatenate(
        [w2t, w_sc.reshape(co, cin).T * s1[:, None], jnp.zeros((cin, co))],
        axis=0).astype(bf16)

    body = functools.partial(_block_body, nb=nb, ho=ho, wo=wo, cin=cin, co=co)
    out = pl.pallas_call(
        body,
        grid=(n // nb,),
        in_specs=[
            pl.BlockSpec((nb, h, w, cin), lambda i: (i, 0, 0, 0)),
            pl.BlockSpec((1, 2 * cin), lambda i: (0, 0)),
            pl.BlockSpec((12 * cin, co), lambda i: (0, 0)),
            pl.BlockSpec((1, co), lambda i: (0, 0)),
            pl.BlockSpec((1, co), lambda i: (0, 0)),
            pl.BlockSpec((9 * co + 2 * cin, co), lambda i: (0, 0)),
        ],
        out_specs=pl.BlockSpec((nb, ho, wo, co), lambda i: (i, 0, 0, 0)),
        out_shape=jax.ShapeDtypeStruct((n, ho, wo, co), jnp.float32),
        compiler_params=pltpu.CompilerParams(
            dimension_semantics=("parallel",),
            vmem_limit_bytes=_VMEM_LIMIT),
        cost_estimate=pl.CostEstimate(
            flops=2 * n * ho * wo * 9 * (cin + co) * co,
            transcendentals=0,
            bytes_accessed=2 * n * h * w * cin + 4 * n * ho * wo * co),
    )(xnh, b1t, w1k, s2.reshape(1, co), b2.reshape(1, co), w2k)

    return jnp.transpose(out, (0, 3, 1, 2))


# speedup vs baseline: 1.0265x; 1.0030x over previous
import functools

import jax
import jax.numpy as jnp
from jax.experimental import pallas as pl
from jax.experimental.pallas import tpu as pltpu

_EPS = 1e-5
_VMEM_LIMIT = 48 * 1024 * 1024


def _one_image(a_all, s2, b2, w1_ref, w2_ref, o_ref, *, ho, wo, cin, co):
    m = ho * wo
    f32 = jnp.float32
    c2 = 2 * cin

    ar = a_all.reshape(ho, 2, wo, c2)
    ev = ar[:, 0]
    od = ar[:, 1]

    p0 = jnp.concatenate([jnp.zeros((1, wo, c2), a_all.dtype),
                          od[:ho - 1]], axis=0)
    planes = (p0, ev, od)

    pieces = []
    for dy in range(3):
        p = planes[dy]
        shift = jnp.concatenate(
            [jnp.zeros((ho, 1, c2), a_all.dtype), p[:, :wo - 1]], axis=1)
        pieces.append(p.reshape(m, c2))
        pieces.append(shift.reshape(m, c2))
    acc = jnp.dot(jnp.concatenate(pieces, axis=1), w1_ref[...],
                  preferred_element_type=f32)

    a2 = jnp.maximum(acc * s2 + b2, 0.0).astype(jnp.bfloat16)
    a2p = jnp.pad(a2.reshape(ho, wo, co), ((1, 1), (1, 1), (0, 0)))

    pieces = []
    for dy in range(3):
        rows = a2p[dy:dy + ho]
        for dx in range(3):
            pieces.append(rows[:, dx:dx + wo].reshape(m, co))
    pieces.append(ev.reshape(m, c2))
    out = jnp.dot(jnp.concatenate(pieces, axis=1), w2_ref[...],
                  preferred_element_type=f32)
    o_ref[...] = out.reshape(ho, wo, co)


def _deint(x_ref):
    xi = pltpu.bitcast(x_ref[...], jnp.int32)
    lo = jax.lax.bitcast_convert_type(xi.astype(jnp.int16), jnp.bfloat16)
    hi = jax.lax.bitcast_convert_type(
        (xi >> 16).astype(jnp.int16), jnp.bfloat16)
    return jnp.concatenate([lo, hi], axis=-1)


def _block_body(x_ref, b1_ref, w1_ref, s2_ref, b2_ref, w2_ref,
                o_ref, *, nb, ho, wo, cin, co):
    xp = _deint(x_ref)

    a = jnp.maximum(xp + b1_ref[0], jnp.bfloat16(0))
    s2, b2 = s2_ref[0], b2_ref[0]
    for b in range(nb):
        _one_image(a[b], s2, b2, w1_ref, w2_ref, o_ref.at[b],
                   ho=ho, wo=wo, cin=cin, co=co)


def kernel(x, bn1_gamma, bn1_beta, bn1_mean, bn1_var,
           bn2_gamma, bn2_beta, bn2_mean, bn2_var, w1, w2, w_sc):
    n, cin, h, w = x.shape
    co = w1.shape[0]
    ho, wo = h // 2, w // 2
    nb = 16 if n % 16 == 0 else 1
    bf16 = jnp.bfloat16

    s1 = bn1_gamma / jnp.sqrt(bn1_var + _EPS)
    b1 = bn1_beta - bn1_mean * s1
    s2 = bn2_gamma / jnp.sqrt(bn2_var + _EPS)
    b2 = bn2_beta - bn2_mean * s2

    xnh = x.transpose(0, 2, 3, 1).astype(bf16)

    b1s = b1 / s1
    b1t = jnp.concatenate([b1s, b1s]).reshape(1, 2 * cin).astype(bf16)

    zero = jnp.zeros((3, cin, co), jnp.float32)
    wt = jnp.transpose(w1, (2, 1, 3, 0)) * s1[None, :, None, None]
    w1k = jnp.concatenate(
        [wt[:, :, 1], wt[:, :, 2], zero, wt[:, :, 0]],
        axis=1).reshape(3 * 4 * cin, co).astype(bf16)

    w2t = jnp.transpose(w2, (2, 3, 1, 0)).reshape(9 * co, co)
    w2k = jnp.concatenate(
        [w2t, w_sc.reshape(co, cin).T * s1[:, None], jnp.zeros((cin, co))],
        axis=0).astype(bf16)

    body = functools.partial(_block_body, nb=nb, ho=ho, wo=wo, cin=cin, co=co)
    out = pl.pallas_call(
        body,
        grid=(n // nb,),
        in_specs=[
            pl.BlockSpec((nb, h, w, cin), lambda i: (i, 0, 0, 0)),
            pl.BlockSpec((1, 2 * cin), lambda i: (0, 0)),
            pl.BlockSpec((12 * cin, co), lambda i: (0, 0)),
            pl.BlockSpec((1, co), lambda i: (0, 0)),
            pl.BlockSpec((1, co), lambda i: (0, 0)),
            pl.BlockSpec((9 * co + 2 * cin, co), lambda i: (0, 0)),
        ],
        out_specs=pl.BlockSpec((nb, ho, wo, co), lambda i: (i, 0, 0, 0)),
        out_shape=jax.ShapeDtypeStruct((n, ho, wo, co), jnp.float32),
        compiler_params=pltpu.CompilerParams(
            dimension_semantics=("parallel",),
            vmem_limit_bytes=_VMEM_LIMIT),
        cost_estimate=pl.CostEstimate(
            flops=2 * n * ho * wo * 9 * (cin + co) * co,
            transcendentals=0,
            bytes_accessed=2 * n * h * w * cin + 4 * n * ho * wo * co),
    )(xnh, b1t, w1k, s2.reshape(1, co), b2.reshape(1, co), w2k)

    return jnp.transpose(out, (0, 3, 1, 2))
